# Initial kernel scaffold; baseline (speedup 1.0000x reference)
import jax
import jax.numpy as jnp
from jax.experimental import pallas as pl


def kernel(x, edge_index, edge_attr, batch, super_batch, mask, params):
    raise NotImplementedError("write your pallas kernel here")


# speedup vs baseline: 1.1124x; 1.1124x over previous
import functools

import jax
import jax.numpy as jnp
from jax.experimental import pallas as pl
from jax.experimental.pallas import tpu as pltpu

N_NODES = 10000
N_EDGES = 160000
N_GRAPHS = 512
N_SUPER = 256
EMB = 300
TEMP = 0.04

EDGE_BLK = 2000
NODE_BLK = 400


def _prec(hi):
    return jax.lax.Precision.HIGHEST if hi else jax.lax.Precision.DEFAULT


def _edge_mlp_body(ea_ref, we_ref, be_ref, out_ref, *, hi):
    out_ref[...] = jax.nn.relu(
        jnp.dot(ea_ref[...], we_ref[...], preferred_element_type=jnp.float32,
                precision=_prec(hi))
        + be_ref[...]
    )


def _edge_mlp_p(edge_attr, We, be, hi):
    d_edge = edge_attr.shape[1]
    return pl.pallas_call(
        functools.partial(_edge_mlp_body, hi=hi),
        grid=(N_EDGES // EDGE_BLK,),
        in_specs=[
            pl.BlockSpec((EDGE_BLK, d_edge), lambda i: (i, 0)),
            pl.BlockSpec((d_edge, EMB), lambda i: (0, 0)),
            pl.BlockSpec((1, EMB), lambda i: (0, 0)),
        ],
        out_specs=pl.BlockSpec((EDGE_BLK, EMB), lambda i: (i, 0)),
        out_shape=jax.ShapeDtypeStruct((N_EDGES, EMB), jnp.float32),
    )(edge_attr, We, be.reshape(1, EMB))


def _edge_mlp(edge_attr, We, be):
    return _edge_mlp_p(edge_attr, We, be, False)


def _node_mlp_body(h_ref, msg_ref, w1_ref, b1_ref, w2_ref, b2_ref, s_ref,
                   out_ref, *, final, hi1, hi2):
    u = h_ref[...] * s_ref[0, 0] + msg_ref[...]
    mid = jax.nn.relu(
        jnp.dot(u, w1_ref[...], preferred_element_type=jnp.float32,
                precision=_prec(hi1))
        + b1_ref[...]
    )
    z = jnp.dot(mid, w2_ref[...], preferred_element_type=jnp.float32,
                precision=_prec(hi2)) + b2_ref[...]
    out_ref[...] = z if final else jax.nn.relu(z)


def _node_mlp(h, msg, p, final):
    return _node_mlp_p(h, msg, p, final, False, False)


def _split2(a):
    hi = a.astype(jnp.bfloat16).astype(jnp.float32)
    lo = (a - hi).astype(jnp.bfloat16).astype(jnp.float32)
    return hi, lo


def _dot_bf16x3(a, w, order):
    ah, al = _split2(a)
    wh, wl = _split2(w)
    d = lambda p, q: jnp.dot(p, q, preferred_element_type=jnp.float32)
    if order == 0:
        return d(ah, wh) + d(ah, wl) + d(al, wh)
    if order == 1:
        return d(ah, wh) + d(al, wh) + d(ah, wl)
    if order == 2:
        return (d(ah, wl) + d(al, wh)) + d(ah, wh)
    return d(ah, wh) + (d(ah, wl) + d(al, wh))


def _node_mlp_3x_body(h_ref, msg_ref, w1_ref, b1_ref, w2_ref, b2_ref, s_ref,
                      out_ref, *, final, order):
    u = h_ref[...] * s_ref[0, 0] + msg_ref[...]
    mid = jax.nn.relu(_dot_bf16x3(u, w1_ref[...], order) + b1_ref[...])
    z = _dot_bf16x3(mid, w2_ref[...], order) + b2_ref[...]
    out_ref[...] = z if final else jax.nn.relu(z)


def _node_mlp_3x(h, msg, p, final, order):
    return pl.pallas_call(
        functools.partial(_node_mlp_3x_body, final=final, order=order),
        grid=(N_NODES // NODE_BLK,),
        in_specs=[
            pl.BlockSpec((NODE_BLK, EMB), lambda i: (i, 0)),
            pl.BlockSpec((NODE_BLK, EMB), lambda i: (i, 0)),
            pl.BlockSpec((EMB, 2 * EMB), lambda i: (0, 0)),
            pl.BlockSpec((1, 2 * EMB), lambda i: (0, 0)),
            pl.BlockSpec((2 * EMB, EMB), lambda i: (0, 0)),
            pl.BlockSpec((1, EMB), lambda i: (0, 0)),
            pl.BlockSpec((1, 1), lambda i: (0, 0)),
        ],
        out_specs=pl.BlockSpec((NODE_BLK, EMB), lambda i: (i, 0)),
        out_shape=jax.ShapeDtypeStruct((N_NODES, EMB), jnp.float32),
    )(h, msg, p['W1'], p['b1'].reshape(1, -1), p['W2'], p['b2'].reshape(1, -1),
      (1.0 + p['eps']).reshape(1, 1))


def _dot_ksplit(a, w, splits):
    acc = None
    off = 0
    for k in splits:
        part = jnp.dot(a[:, off:off + k], w[off:off + k, :],
                       preferred_element_type=jnp.float32)
        acc = part if acc is None else acc + part
        off += k
    return acc


def _node_mlp_ks_body(h_ref, msg_ref, w1_ref, b1_ref, w2_ref, b2_ref, s_ref,
                      out_ref, *, final, ks1, ks2):
    u = h_ref[...] * s_ref[0, 0] + msg_ref[...]
    mid = jax.nn.relu(_dot_ksplit(u, w1_ref[...], ks1) + b1_ref[...])
    z = _dot_ksplit(mid, w2_ref[...], ks2) + b2_ref[...]
    out_ref[...] = z if final else jax.nn.relu(z)


def _node_mlp_ks(h, msg, p, final, ks1, ks2):
    return pl.pallas_call(
        functools.partial(_node_mlp_ks_body, final=final, ks1=ks1, ks2=ks2),
        grid=(N_NODES // NODE_BLK,),
        in_specs=[
            pl.BlockSpec((NODE_BLK, EMB), lambda i: (i, 0)),
            pl.BlockSpec((NODE_BLK, EMB), lambda i: (i, 0)),
            pl.BlockSpec((EMB, 2 * EMB), lambda i: (0, 0)),
            pl.BlockSpec((1, 2 * EMB), lambda i: (0, 0)),
            pl.BlockSpec((2 * EMB, EMB), lambda i: (0, 0)),
            pl.BlockSpec((1, EMB), lambda i: (0, 0)),
            pl.BlockSpec((1, 1), lambda i: (0, 0)),
        ],
        out_specs=pl.BlockSpec((NODE_BLK, EMB), lambda i: (i, 0)),
        out_shape=jax.ShapeDtypeStruct((N_NODES, EMB), jnp.float32),
    )(h, msg, p['W1'], p['b1'].reshape(1, -1), p['W2'], p['b2'].reshape(1, -1),
      (1.0 + p['eps']).reshape(1, 1))


def _node_mlp_p(h, msg, p, final, hi1, hi2):
    return pl.pallas_call(
        functools.partial(_node_mlp_body, final=final, hi1=hi1, hi2=hi2),
        grid=(N_NODES // NODE_BLK,),
        in_specs=[
            pl.BlockSpec((NODE_BLK, EMB), lambda i: (i, 0)),
            pl.BlockSpec((NODE_BLK, EMB), lambda i: (i, 0)),
            pl.BlockSpec((EMB, 2 * EMB), lambda i: (0, 0)),
            pl.BlockSpec((1, 2 * EMB), lambda i: (0, 0)),
            pl.BlockSpec((2 * EMB, EMB), lambda i: (0, 0)),
            pl.BlockSpec((1, EMB), lambda i: (0, 0)),
            pl.BlockSpec((1, 1), lambda i: (0, 0)),
        ],
        out_specs=pl.BlockSpec((NODE_BLK, EMB), lambda i: (i, 0)),
        out_shape=jax.ShapeDtypeStruct((N_NODES, EMB), jnp.float32),
    )(h, msg, p['W1'], p['b1'].reshape(1, -1), p['W2'], p['b2'].reshape(1, -1),
      (1.0 + p['eps']).reshape(1, 1))


def _pool_body(batch_ref, h_ref, out_ref, acc_ref, cnt_ref):
    i = pl.program_id(0)

    @pl.when(i == 0)
    def _():
        acc_ref[...] = jnp.zeros_like(acc_ref)
        cnt_ref[...] = jnp.zeros_like(cnt_ref)

    seg = batch_ref[0, 0, :]
    gid = jax.lax.broadcasted_iota(jnp.int32, (N_GRAPHS, NODE_BLK), 0)
    S = (seg[None, :] == gid).astype(jnp.float32)
    acc_ref[...] += jnp.dot(S, h_ref[...], preferred_element_type=jnp.float32, precision=jax.lax.Precision.HIGHEST)
    cnt_ref[...] += jnp.sum(S, axis=1, keepdims=True)

    @pl.when(i == pl.num_programs(0) - 1)
    def _():
        out_ref[...] = acc_ref[...] / jnp.maximum(cnt_ref[...], 1.0)


def _mean_pool_graphs(h, batch):
    batch3 = batch.astype(jnp.int32).reshape(N_NODES // NODE_BLK, 1, NODE_BLK)
    return pl.pallas_call(
        _pool_body,
        grid=(N_NODES // NODE_BLK,),
        in_specs=[
            pl.BlockSpec((1, 1, NODE_BLK), lambda i: (i, 0, 0)),
            pl.BlockSpec((NODE_BLK, EMB), lambda i: (i, 0)),
        ],
        out_specs=pl.BlockSpec((N_GRAPHS, EMB), lambda i: (0, 0)),
        out_shape=jax.ShapeDtypeStruct((N_GRAPHS, EMB), jnp.float32),
        scratch_shapes=[
            pltpu.VMEM((N_GRAPHS, EMB), jnp.float32),
            pltpu.VMEM((N_GRAPHS, 1), jnp.float32),
        ],
    )(batch3, h)


def _head_body(out_ref, maskf_ref, sb_ref, pw1_ref, pb1_ref, pw2_ref, pb2_ref,
               sw1_ref, sb1_ref, sw2_ref, sb2_ref, logits_ref, *,
               mlp_hi, logit_hi):
    out = out_ref[...]
    maskf = maskf_ref[...]
    sb = sb_ref[0, 0, :]

    def mlp2(x, w1, b1, w2, b2):
        mid = jax.nn.relu(jnp.dot(x, w1, preferred_element_type=jnp.float32,
                                  precision=_prec(mlp_hi)) + b1)
        return jnp.dot(mid, w2, preferred_element_type=jnp.float32,
                       precision=_prec(mlp_hi)) + b2

    def l2norm(x):
        n = jnp.sqrt(jnp.sum(x * x, axis=1, keepdims=True))
        return x / jnp.maximum(n, 1e-12)

    gsel = jax.lax.broadcasted_iota(jnp.int32, (N_SUPER, N_GRAPHS), 0) * 2
    gall = jax.lax.broadcasted_iota(jnp.int32, (N_SUPER, N_GRAPHS), 1)
    Psel = (gsel == gall).astype(jnp.float32)
    frag = jnp.dot(Psel * maskf, out, preferred_element_type=jnp.float32, precision=jax.lax.Precision.HIGHEST)
    target = l2norm(mlp2(frag, pw1_ref[...], pb1_ref[...], pw2_ref[...], pb2_ref[...]))

    sid = jax.lax.broadcasted_iota(jnp.int32, (N_SUPER, N_GRAPHS), 0)
    Ssup = (sb[None, :] == sid).astype(jnp.float32)
    cnt = jnp.sum(Ssup, axis=1, keepdims=True)
    pooled = jnp.dot(Ssup * (1.0 - maskf), out,
                     preferred_element_type=jnp.float32, precision=jax.lax.Precision.HIGHEST) / jnp.maximum(cnt, 1.0)
    pred = l2norm(mlp2(pooled, sw1_ref[...], sb1_ref[...], sw2_ref[...], sb2_ref[...]))

    logits_ref[...] = jnp.dot(pred, target.T,
                              preferred_element_type=jnp.float32,
                              precision=_prec(logit_hi)) * (1.0 / TEMP)


def _head(out, mask, super_batch, proj, sup, mlp_hi=False, logit_hi=False):
    maskf = mask.astype(jnp.float32).reshape(1, N_GRAPHS)
    sb3 = super_batch.astype(jnp.int32).reshape(1, 1, N_GRAPHS)
    full = lambda s: pl.BlockSpec(s, lambda: tuple(0 for _ in s))
    return pl.pallas_call(
        functools.partial(_head_body, mlp_hi=mlp_hi, logit_hi=logit_hi),
        in_specs=[
            full((N_GRAPHS, EMB)),
            full((1, N_GRAPHS)),
            full((1, 1, N_GRAPHS)),
            full((EMB, EMB)), full((1, EMB)), full((EMB, EMB)), full((1, EMB)),
            full((EMB, EMB)), full((1, EMB)), full((EMB, EMB)), full((1, EMB)),
        ],
        out_specs=full((N_SUPER, N_SUPER)),
        out_shape=jax.ShapeDtypeStruct((N_SUPER, N_SUPER), jnp.float32),
    )(out, maskf, sb3,
      proj['W1'], proj['b1'].reshape(1, -1), proj['W2'], proj['b2'].reshape(1, -1),
      sup['W1'], sup['b1'].reshape(1, -1), sup['W2'], sup['b2'].reshape(1, -1))


def kernel(x, edge_index, edge_attr, batch, super_batch, mask, params):
    src, dst = edge_index[0], edge_index[1]
    h = x
    n_layers = len(params['layers'])
    for l, p in enumerate(params['layers']):
        E = _edge_mlp(edge_attr, p['We'], p['be'])
        msg = jax.ops.segment_sum(h[src] + E, dst, num_segments=N_NODES)
        h = _node_mlp(h, msg, p, final=(l == n_layers - 1))
    out = _mean_pool_graphs(h, batch)
    logits = _head(out, mask, super_batch, params['proj'], params['super'])
    labels = jnp.arange(N_SUPER, dtype=jnp.int32)
    return logits, labels


# speedup vs baseline: 1.1819x; 1.0625x over previous
import functools

import jax
import jax.numpy as jnp
from jax import lax
from jax.experimental import pallas as pl
from jax.experimental.pallas import tpu as pltpu
from jax.experimental.pallas import tpu_sc as plsc

N_NODES = 10000
N_EDGES = 160000
N_GRAPHS = 512
N_SUPER = 256
EMB = 300
TEMP = 0.04

EDGE_BLK = 2000
NODE_BLK = 400

EMBP = 384
QROWS = 2504
MSG_ROWS = 4 * QROWS
ACC_ROWS = 2560
EB = 80
EPT = N_EDGES // 16


EPW = N_EDGES // 32


def _sc_gather_body(h_hbm, src_hbm, g_hbm, srcv, rows, rows2, sem):
    c = lax.axis_index("c")
    s = lax.axis_index("s")
    w = c * 16 + s

    def body(j, carry):
        base = w * EPW + j * (2 * EB)
        pltpu.sync_copy(src_hbm.at[pl.ds(base, 2 * EB)], srcv)
        cp1 = pltpu.async_copy(h_hbm.at[srcv.at[pl.ds(0, EB)]], rows, sem)
        cp2 = pltpu.async_copy(h_hbm.at[srcv.at[pl.ds(EB, EB)]], rows2, sem)
        cp1.wait()
        pltpu.sync_copy(rows, g_hbm.at[pl.ds(base, EB)])
        cp2.wait()
        pltpu.sync_copy(rows2, g_hbm.at[pl.ds(base + EB, EB)])
        return carry

    lax.fori_loop(0, EPW // (2 * EB), body, 0)


def _sc_gather(h_pad, src):
    mesh = plsc.VectorSubcoreMesh(core_axis_name="c", subcore_axis_name="s")
    k = functools.partial(
        pl.kernel,
        out_type=jax.ShapeDtypeStruct((N_EDGES, EMBP), jnp.float32),
        mesh=mesh,
        scratch_types=[
            pltpu.VMEM((2 * EB,), jnp.int32),
            pltpu.VMEM((EB, EMBP), jnp.float32),
            pltpu.VMEM((EB, EMBP), jnp.float32),
            pltpu.SemaphoreType.DMA,
        ],
    )(_sc_gather_body)
    return k(h_pad, src)


def _prec(hi):
    return jax.lax.Precision.HIGHEST if hi else jax.lax.Precision.DEFAULT


def _edge_mlp_body(ea_ref, we_ref, be_ref, out_ref, *, hi):
    out_ref[...] = jax.nn.relu(
        jnp.dot(ea_ref[...], we_ref[...], preferred_element_type=jnp.float32,
                precision=_prec(hi))
        + be_ref[...]
    )


def _edge_mlp_p(edge_attr, We, be, hi):
    d_edge = edge_attr.shape[1]
    w = We.shape[1]
    return pl.pallas_call(
        functools.partial(_edge_mlp_body, hi=hi),
        grid=(N_EDGES // EDGE_BLK,),
        in_specs=[
            pl.BlockSpec((EDGE_BLK, d_edge), lambda i: (i, 0)),
            pl.BlockSpec((d_edge, w), lambda i: (0, 0)),
            pl.BlockSpec((1, w), lambda i: (0, 0)),
        ],
        out_specs=pl.BlockSpec((EDGE_BLK, w), lambda i: (i, 0)),
        out_shape=jax.ShapeDtypeStruct((N_EDGES, w), jnp.float32),
    )(edge_attr, We, be.reshape(1, w))


def _edge_mlp(edge_attr, We, be):
    return _edge_mlp_p(edge_attr, We, be, False)


def _node_mlp_body(h_ref, msg_ref, w1_ref, b1_ref, w2_ref, b2_ref,
                   s_ref, out_ref, *, final, hi1, hi2):
    u = h_ref[...] * s_ref[0, 0] + msg_ref[...]
    mid = jax.nn.relu(
        jnp.dot(u, w1_ref[...], preferred_element_type=jnp.float32,
                precision=_prec(hi1))
        + b1_ref[...]
    )
    z = jnp.dot(mid, w2_ref[...], preferred_element_type=jnp.float32,
                precision=_prec(hi2)) + b2_ref[...]
    out_ref[...] = z if final else jax.nn.relu(z)


def _node_mlp(h, msg, p, final):
    return _node_mlp_p(h, msg, p, final, False, False)


def _split2(a):
    hi = a.astype(jnp.bfloat16).astype(jnp.float32)
    lo = (a - hi).astype(jnp.bfloat16).astype(jnp.float32)
    return hi, lo


def _dot_bf16x3(a, w, order):
    ah, al = _split2(a)
    wh, wl = _split2(w)
    d = lambda p, q: jnp.dot(p, q, preferred_element_type=jnp.float32)
    if order == 0:
        return d(ah, wh) + d(ah, wl) + d(al, wh)
    if order == 1:
        return d(ah, wh) + d(al, wh) + d(ah, wl)
    if order == 2:
        return (d(ah, wl) + d(al, wh)) + d(ah, wh)
    return d(ah, wh) + (d(ah, wl) + d(al, wh))


def _node_mlp_3x_body(h_ref, msg_ref, w1_ref, b1_ref, w2_ref, b2_ref, s_ref,
                      out_ref, *, final, order):
    u = h_ref[...] * s_ref[0, 0] + msg_ref[...]
    mid = jax.nn.relu(_dot_bf16x3(u, w1_ref[...], order) + b1_ref[...])
    z = _dot_bf16x3(mid, w2_ref[...], order) + b2_ref[...]
    out_ref[...] = z if final else jax.nn.relu(z)


def _node_mlp_3x(h, msg, p, final, order):
    return pl.pallas_call(
        functools.partial(_node_mlp_3x_body, final=final, order=order),
        grid=(N_NODES // NODE_BLK,),
        in_specs=[
            pl.BlockSpec((NODE_BLK, EMB), lambda i: (i, 0)),
            pl.BlockSpec((NODE_BLK, EMB), lambda i: (i, 0)),
            pl.BlockSpec((EMB, 2 * EMB), lambda i: (0, 0)),
            pl.BlockSpec((1, 2 * EMB), lambda i: (0, 0)),
            pl.BlockSpec((2 * EMB, EMB), lambda i: (0, 0)),
            pl.BlockSpec((1, EMB), lambda i: (0, 0)),
            pl.BlockSpec((1, 1), lambda i: (0, 0)),
        ],
        out_specs=pl.BlockSpec((NODE_BLK, EMB), lambda i: (i, 0)),
        out_shape=jax.ShapeDtypeStruct((N_NODES, EMB), jnp.float32),
    )(h, msg, p['W1'], p['b1'].reshape(1, -1), p['W2'], p['b2'].reshape(1, -1),
      (1.0 + p['eps']).reshape(1, 1))


def _dot_ksplit(a, w, splits):
    acc = None
    off = 0
    for k in splits:
        part = jnp.dot(a[:, off:off + k], w[off:off + k, :],
                       preferred_element_type=jnp.float32)
        acc = part if acc is None else acc + part
        off += k
    return acc


def _node_mlp_ks_body(h_ref, msg_ref, w1_ref, b1_ref, w2_ref, b2_ref, s_ref,
                      out_ref, *, final, ks1, ks2):
    u = h_ref[...] * s_ref[0, 0] + msg_ref[...]
    mid = jax.nn.relu(_dot_ksplit(u, w1_ref[...], ks1) + b1_ref[...])
    z = _dot_ksplit(mid, w2_ref[...], ks2) + b2_ref[...]
    out_ref[...] = z if final else jax.nn.relu(z)


def _node_mlp_ks(h, msg, p, final, ks1, ks2):
    return pl.pallas_call(
        functools.partial(_node_mlp_ks_body, final=final, ks1=ks1, ks2=ks2),
        grid=(N_NODES // NODE_BLK,),
        in_specs=[
            pl.BlockSpec((NODE_BLK, EMB), lambda i: (i, 0)),
            pl.BlockSpec((NODE_BLK, EMB), lambda i: (i, 0)),
            pl.BlockSpec((EMB, 2 * EMB), lambda i: (0, 0)),
            pl.BlockSpec((1, 2 * EMB), lambda i: (0, 0)),
            pl.BlockSpec((2 * EMB, EMB), lambda i: (0, 0)),
            pl.BlockSpec((1, EMB), lambda i: (0, 0)),
            pl.BlockSpec((1, 1), lambda i: (0, 0)),
        ],
        out_specs=pl.BlockSpec((NODE_BLK, EMB), lambda i: (i, 0)),
        out_shape=jax.ShapeDtypeStruct((N_NODES, EMB), jnp.float32),
    )(h, msg, p['W1'], p['b1'].reshape(1, -1), p['W2'], p['b2'].reshape(1, -1),
      (1.0 + p['eps']).reshape(1, 1))


def _node_mlp_p(h, msg, p, final, hi1, hi2):
    win = p['W1'].shape[0]
    wout = p['W2'].shape[1]
    return pl.pallas_call(
        functools.partial(_node_mlp_body, final=final, hi1=hi1, hi2=hi2),
        grid=(N_NODES // NODE_BLK,),
        in_specs=[
            pl.BlockSpec((NODE_BLK, win), lambda i: (i, 0)),
            pl.BlockSpec((NODE_BLK, win), lambda i: (i, 0)),
            pl.BlockSpec((win, 2 * EMB), lambda i: (0, 0)),
            pl.BlockSpec((1, 2 * EMB), lambda i: (0, 0)),
            pl.BlockSpec((2 * EMB, wout), lambda i: (0, 0)),
            pl.BlockSpec((1, wout), lambda i: (0, 0)),
            pl.BlockSpec((1, 1), lambda i: (0, 0)),
        ],
        out_specs=pl.BlockSpec((NODE_BLK, wout), lambda i: (i, 0)),
        out_shape=jax.ShapeDtypeStruct((N_NODES, wout), jnp.float32),
    )(h, msg, p['W1'], p['b1'].reshape(1, -1), p['W2'],
      p['b2'].reshape(1, -1), (1.0 + p['eps']).reshape(1, 1))


def _pool_body(batch_ref, h_ref, out_ref, acc_ref, cnt_ref):
    i = pl.program_id(0)

    @pl.when(i == 0)
    def _():
        acc_ref[...] = jnp.zeros_like(acc_ref)
        cnt_ref[...] = jnp.zeros_like(cnt_ref)

    seg = batch_ref[0, 0, :]
    gid = jax.lax.broadcasted_iota(jnp.int32, (N_GRAPHS, NODE_BLK), 0)
    S = (seg[None, :] == gid).astype(jnp.float32)
    acc_ref[...] += jnp.dot(S, h_ref[...], preferred_element_type=jnp.float32, precision=jax.lax.Precision.HIGHEST)
    cnt_ref[...] += jnp.sum(S, axis=1, keepdims=True)

    @pl.when(i == pl.num_programs(0) - 1)
    def _():
        out_ref[...] = acc_ref[...] / jnp.maximum(cnt_ref[...], 1.0)


def _mean_pool_graphs(h, batch):
    w = h.shape[1]
    batch3 = batch.astype(jnp.int32).reshape(N_NODES // NODE_BLK, 1, NODE_BLK)
    return pl.pallas_call(
        _pool_body,
        grid=(N_NODES // NODE_BLK,),
        in_specs=[
            pl.BlockSpec((1, 1, NODE_BLK), lambda i: (i, 0, 0)),
            pl.BlockSpec((NODE_BLK, w), lambda i: (i, 0)),
        ],
        out_specs=pl.BlockSpec((N_GRAPHS, w), lambda i: (0, 0)),
        out_shape=jax.ShapeDtypeStruct((N_GRAPHS, w), jnp.float32),
        scratch_shapes=[
            pltpu.VMEM((N_GRAPHS, w), jnp.float32),
            pltpu.VMEM((N_GRAPHS, 1), jnp.float32),
        ],
    )(batch3, h)


def _head_body(out_ref, maskf_ref, sb_ref, pw1_ref, pb1_ref, pw2_ref, pb2_ref,
               sw1_ref, sb1_ref, sw2_ref, sb2_ref, logits_ref, *,
               mlp_hi, logit_hi):
    out = out_ref[...]
    maskf = maskf_ref[...]
    sb = sb_ref[0, 0, :]

    def mlp2(x, w1, b1, w2, b2):
        mid = jax.nn.relu(jnp.dot(x, w1, preferred_element_type=jnp.float32,
                                  precision=_prec(mlp_hi)) + b1)
        return jnp.dot(mid, w2, preferred_element_type=jnp.float32,
                       precision=_prec(mlp_hi)) + b2

    def l2norm(x):
        n = jnp.sqrt(jnp.sum(x * x, axis=1, keepdims=True))
        return x / jnp.maximum(n, 1e-12)

    gsel = jax.lax.broadcasted_iota(jnp.int32, (N_SUPER, N_GRAPHS), 0) * 2
    gall = jax.lax.broadcasted_iota(jnp.int32, (N_SUPER, N_GRAPHS), 1)
    Psel = (gsel == gall).astype(jnp.float32)
    frag = jnp.dot(Psel * maskf, out, preferred_element_type=jnp.float32, precision=jax.lax.Precision.HIGHEST)
    target = l2norm(mlp2(frag, pw1_ref[...], pb1_ref[...], pw2_ref[...], pb2_ref[...]))

    sid = jax.lax.broadcasted_iota(jnp.int32, (N_SUPER, N_GRAPHS), 0)
    Ssup = (sb[None, :] == sid).astype(jnp.float32)
    cnt = jnp.sum(Ssup, axis=1, keepdims=True)
    pooled = jnp.dot(Ssup * (1.0 - maskf), out,
                     preferred_element_type=jnp.float32, precision=jax.lax.Precision.HIGHEST) / jnp.maximum(cnt, 1.0)
    pred = l2norm(mlp2(pooled, sw1_ref[...], sb1_ref[...], sw2_ref[...], sb2_ref[...]))

    logits_ref[...] = jnp.dot(pred, target.T,
                              preferred_element_type=jnp.float32,
                              precision=_prec(logit_hi)) * (1.0 / TEMP)


def _head(out, mask, super_batch, proj, sup, mlp_hi=False, logit_hi=False):
    maskf = mask.astype(jnp.float32).reshape(1, N_GRAPHS)
    sb3 = super_batch.astype(jnp.int32).reshape(1, 1, N_GRAPHS)
    full = lambda s: pl.BlockSpec(s, lambda: tuple(0 for _ in s))
    return pl.pallas_call(
        functools.partial(_head_body, mlp_hi=mlp_hi, logit_hi=logit_hi),
        in_specs=[
            full((N_GRAPHS, EMB)),
            full((1, N_GRAPHS)),
            full((1, 1, N_GRAPHS)),
            full((EMB, EMB)), full((1, EMB)), full((EMB, EMB)), full((1, EMB)),
            full((EMB, EMB)), full((1, EMB)), full((EMB, EMB)), full((1, EMB)),
        ],
        out_specs=full((N_SUPER, N_SUPER)),
        out_shape=jax.ShapeDtypeStruct((N_SUPER, N_SUPER), jnp.float32),
    )(out, maskf, sb3,
      proj['W1'], proj['b1'].reshape(1, -1), proj['W2'], proj['b2'].reshape(1, -1),
      sup['W1'], sup['b1'].reshape(1, -1), sup['W2'], sup['b2'].reshape(1, -1))


def kernel(x, edge_index, edge_attr, batch, super_batch, mask, params):
    src = edge_index[0].astype(jnp.int32)
    dst = edge_index[1].astype(jnp.int32)
    pad = EMBP - EMB
    h = jnp.pad(x, ((0, 0), (0, pad)))
    n_layers = len(params['layers'])
    for l, p in enumerate(params['layers']):
        We_p = jnp.pad(p['We'], ((0, 0), (0, pad)))
        be_p = jnp.pad(p['be'], (0, pad))
        E = _edge_mlp(edge_attr, We_p, be_p)
        G = _sc_gather(h, src)
        msg = jax.ops.segment_sum(G + E, dst, num_segments=N_NODES)
        pp = {'W1': jnp.pad(p['W1'], ((0, pad), (0, 0))), 'b1': p['b1'],
              'W2': jnp.pad(p['W2'], ((0, 0), (0, pad))),
              'b2': jnp.pad(p['b2'], (0, pad)), 'eps': p['eps']}
        h = _node_mlp(h, msg, pp, final=(l == n_layers - 1))
    out = _mean_pool_graphs(h, batch)[:, :EMB]
    logits = _head(out, mask, super_batch, params['proj'], params['super'])
    labels = jnp.arange(N_SUPER, dtype=jnp.int32)
    return logits, labels
